# baseline (speedup 1.0000x reference)
import functools
import math

import jax
import jax.numpy as jnp
from jax import lax
from jax.experimental import pallas as pl
from jax.experimental.pallas import tpu as pltpu
from jax.experimental.pallas import tpu_sc as plsc

N_NODES = 19
WINDOW = 100
STRIDE = 50


def _sc_win_kernel(xflat, out, inb0, inb1, outb0, outb1, isem0, isem1,
                   osem0, osem1, *, B, T):
    N = N_NODES
    W = (T - WINDOW) // STRIDE + 1
    SN = STRIDE * N
    TOTAL = B * T * N
    CW = 8
    RWS = CW * N
    NCHUNK = (B * W) // CW
    NWK = 32
    NI = -(-NCHUNK // NWK)
    assert NI % 2 == 0
    ILEN = inb0.shape[0]
    CLAMP = TOTAL - ILEN
    slots = ((inb0, outb0, isem0, osem0), (inb1, outb1, isem1, osem1))

    wid = lax.axis_index("s") * 2 + lax.axis_index("c")

    def chunk_src(c):
        gw0 = c * CW
        q = gw0 // W
        s0 = q * (T * N) + (gw0 - q * W) * SN
        s8 = pl.multiple_of(jnp.minimum(s0 - lax.rem(s0, 8), CLAMP), 8)
        return gw0, s8

    def copy_in(c, sl):
        _, s8 = chunk_src(c)
        return pltpu.make_async_copy(
            xflat.at[pl.ds(s8, ILEN)], slots[sl][0], slots[sl][2]
        )

    def copy_out(c, sl):
        return pltpu.make_async_copy(
            slots[sl][1], out.at[pl.ds(c * RWS, RWS)], slots[sl][3]
        )

    cols = (0, 16, 32, 48, 64, 80, WINDOW - 16)
    lane = lax.iota(jnp.int32, 16)
    gvecs = [lane * N + c * N for c in cols]

    copy_in(wid, 0).start()
    copy_in(wid + NWK, 1).start()

    def unit(i, carry):
        for sl in range(2):
            u = 2 * i + sl
            c = wid + u * NWK
            live = c < NCHUNK
            src, dst = slots[sl][0], slots[sl][1]

            @pl.when(live)
            def _():
                copy_in(c, sl).wait()

            @pl.when((u >= 2) & live)
            def _():
                copy_out(c - 2 * NWK, sl).wait()

            gw0, s8 = chunk_src(c)

            def row_body(r, rcarry, gw0=gw0, s8=s8, src=src, dst=dst):
                wloc = r // N
                n = r - wloc * N
                gwr = gw0 + wloc
                qr = gwr // W
                sr = qr * (T * N) + (gwr - qr * W) * SN
                base = sr - s8 + n
                for g in range(len(cols)):
                    dst[r, pl.ds(cols[g], 16)] = plsc.load_gather(
                        src, [gvecs[g] + base]
                    )
                return rcarry

            @pl.when(live)
            def _():
                lax.fori_loop(0, RWS, row_body, 0, unroll=2)
                copy_out(c, sl).start()

            @pl.when(c + 2 * NWK < NCHUNK)
            def _():
                copy_in(c + 2 * NWK, sl).start()

        return carry

    lax.fori_loop(0, NI // 2, unit, 0)

    for k in range(2):
        tail = wid + (NI - 2 + k) * NWK

        @pl.when(tail < NCHUNK)
        def _(tail=tail, k=k):
            copy_out(tail, (NI - 2 + k) % 2).wait()


def _edge_kernel(pre_ref, ewrep_ref, eib_ref, ewb_ref, goff: int):
    j = pl.program_id(0)
    eib_ref[...] = pre_ref[...] + j * goff
    ewb_ref[...] = ewrep_ref[...]


def _bv_kernel(bv_ref):
    r = jax.lax.broadcasted_iota(jnp.int32, bv_ref.shape, 0)
    bv_ref[...] = r // N_NODES


def kernel(x, edge_index, edge_weight):
    B, T, N = x.shape
    W = (T - WINDOW) // STRIDE + 1
    G = B * W
    E = edge_index.shape[1]

    xflat = x.reshape(B * T * N)
    mesh = plsc.VectorSubcoreMesh(core_axis_name="c", subcore_axis_name="s")
    CW = 8
    span = CW * STRIDE * N + WINDOW * N
    ILEN = -(-(span + 8) // 8) * 8
    sc_win = pl.kernel(
        functools.partial(_sc_win_kernel, B=B, T=T),
        mesh=mesh,
        compiler_params=pltpu.CompilerParams(needs_layout_passes=False),
        out_type=jax.ShapeDtypeStruct((G * N, WINDOW), jnp.float32),
        scratch_types=[
            pltpu.VMEM((ILEN,), jnp.float32),
            pltpu.VMEM((ILEN,), jnp.float32),
            pltpu.VMEM((CW * N, WINDOW), jnp.float32),
            pltpu.VMEM((CW * N, WINDOW), jnp.float32),
            pltpu.SemaphoreType.DMA,
            pltpu.SemaphoreType.DMA,
            pltpu.SemaphoreType.DMA,
            pltpu.SemaphoreType.DMA,
        ],
    )
    x_batched = sc_win(xflat)

    ei = edge_index.astype(jnp.int32)
    CE = G * E
    lcm = (E * 1024) // math.gcd(E, 1024)
    gstep = lcm // E
    CB = lcm
    nblk = -(-CE // CB)
    col = jnp.arange(CB, dtype=jnp.int32)
    pre = jnp.tile(ei, (1, gstep)) + (col // E * N)[None, :]
    ewrep = jnp.tile(edge_weight, gstep)

    ei_b, ew_b = pl.pallas_call(
        lambda p, w, o1, o2: _edge_kernel(p, w, o1, o2, gstep * N),
        grid=(nblk,),
        in_specs=[
            pl.BlockSpec((2, CB), lambda j: (0, 0)),
            pl.BlockSpec((CB,), lambda j: (0,)),
        ],
        out_specs=[
            pl.BlockSpec((2, CB), lambda j: (0, j)),
            pl.BlockSpec((CB,), lambda j: (j,)),
        ],
        out_shape=[
            jax.ShapeDtypeStruct((2, CE), jnp.int32),
            jax.ShapeDtypeStruct((CE,), jnp.float32),
        ],
    )(pre, ewrep)

    batch_vec = pl.pallas_call(
        _bv_kernel,
        out_shape=jax.ShapeDtypeStruct((G * N,), jnp.int32),
    )()

    return x_batched, ei_b, ew_b, batch_vec
